# baseline (speedup 1.0000x reference)
import functools

import jax
import jax.numpy as jnp
from jax import lax
from jax.experimental import pallas as pl
from jax.experimental.pallas import tpu as pltpu
from jax.experimental.pallas import tpu_sc as plsc

N_EMB = 8192
DIM = 64
ROWS = 8192
BM = 512
N_BLOCKS = ROWS // BM

SC_CORES = 2
SC_SUBCORES = 16
NW = SC_CORES * SC_SUBCORES
B_PER_W = ROWS // NW
IDX_CHUNK = 128
N_CHUNKS = B_PER_W // IDX_CHUNK
DIM_PAD = 128


def _dist_kernel(x_ref, w_ref, dist_ref, enc_ref, loss_ref):
    x = x_ref[...]
    w = w_ref[...]
    xw = lax.dot_general(x, w, (((1,), (1,)), ((), ())),
                         preferred_element_type=jnp.float32)
    x2 = jnp.sum(x * x, axis=1, keepdims=True)
    w2 = jnp.sum(w * w, axis=1)
    d = x2 + w2[None, :] - 2.0 * xw
    dist_ref[...] = d
    m = jnp.full((BM,), jnp.inf, jnp.float32)
    e = jnp.zeros((BM,), jnp.int32)
    CC = 1024
    for c in range(N_EMB // CC):
        dc = d[:, c * CC:(c + 1) * CC]
        mc = jnp.min(dc, axis=1)
        ec = jnp.argmin(dc, axis=1).astype(jnp.int32) + c * CC
        upd = mc < m
        e = jnp.where(upd, ec, e)
        m = jnp.minimum(m, mc)
    enc_ref[...] = e[:, None]
    part = jnp.sum(m)

    @pl.when(pl.program_id(0) == 0)
    def _():
        loss_ref[...] = jnp.zeros_like(loss_ref)

    loss_ref[...] += jnp.full((1, 1), part, jnp.float32)


def _distances_enc_loss(flat_x, weight):
    return pl.pallas_call(
        _dist_kernel,
        grid=(N_BLOCKS,),
        in_specs=[
            pl.BlockSpec((BM, DIM), lambda i: (i, 0)),
            pl.BlockSpec((N_EMB, DIM), lambda i: (0, 0)),
        ],
        out_specs=[
            pl.BlockSpec((BM, N_EMB), lambda i: (i, 0)),
            pl.BlockSpec((BM, 1), lambda i: (i, 0)),
            pl.BlockSpec((1, 1), lambda i: (0, 0)),
        ],
        out_shape=[
            jax.ShapeDtypeStruct((ROWS, N_EMB), jnp.float32),
            jax.ShapeDtypeStruct((ROWS, 1), jnp.int32),
            jax.ShapeDtypeStruct((1, 1), jnp.float32),
        ],
    )(flat_x, weight)


def _sc_gather_body(w_hbm, enc_hbm, out_hbm, idx_v, rows_v, sem):
    wid = lax.axis_index("s") * SC_CORES + lax.axis_index("c")
    base = wid * B_PER_W
    pltpu.sync_copy(enc_hbm.at[pl.ds(wid * N_CHUNKS, N_CHUNKS)], idx_v)
    for j in range(N_CHUNKS):
        pltpu.async_copy(w_hbm.at[idx_v.at[j]],
                         rows_v.at[pl.ds(j * IDX_CHUNK, IDX_CHUNK)], sem).wait()
    pltpu.sync_copy(rows_v, out_hbm.at[pl.ds(base, B_PER_W)])


@functools.cache
def _sc_gather():
    return pl.kernel(
        _sc_gather_body,
        out_type=jax.ShapeDtypeStruct((ROWS, DIM_PAD), jnp.float32),
        scratch_types=[
            pltpu.VMEM((N_CHUNKS, IDX_CHUNK), jnp.int32),
            pltpu.VMEM((B_PER_W, DIM_PAD), jnp.float32),
            pltpu.SemaphoreType.DMA,
        ],
        mesh=plsc.VectorSubcoreMesh(core_axis_name="c", subcore_axis_name="s"),
    )


def kernel(input, weight):
    flat_x = jnp.transpose(input, (0, 2, 3, 1)).reshape(ROWS, DIM)
    distances, enc2d, loss_acc = _distances_enc_loss(flat_x, weight)
    encoding_flat = enc2d.reshape(ROWS)
    weight_pad = jnp.pad(weight, ((0, 0), (0, DIM_PAD - DIM)))
    quant_pad = _sc_gather()(weight_pad,
                             enc2d.reshape(ROWS // IDX_CHUNK, IDX_CHUNK))
    quant_flat = quant_pad[:, :DIM]
    quantized_st = jnp.transpose(
        quant_flat.reshape(8, 32, 32, DIM), (0, 3, 1, 2))
    encoding = encoding_flat.reshape(8, 32, 32)
    loss = loss_acc[0, 0] * (1.0 / input.size)
    return (quantized_st, encoding, distances, loss)


# speedup vs baseline: 1.6943x; 1.6943x over previous
import functools

import jax
import jax.numpy as jnp
from jax import lax
from jax.experimental import pallas as pl
from jax.experimental.pallas import tpu as pltpu
from jax.experimental.pallas import tpu_sc as plsc

N_EMB = 8192
DIM = 64
ROWS = 8192
BM = 512
N_BLOCKS = ROWS // BM
KAUG = 128

SC_CORES = 2
SC_SUBCORES = 16
NW = SC_CORES * SC_SUBCORES
B_PER_W = ROWS // NW
IDX_CHUNK = 128
N_CHUNKS = B_PER_W // IDX_CHUNK
DIM_PAD = 128


def _dist_kernel(x_ref, w_ref, dist_ref, enc_ref, loss_ref, waug_ref):
    @pl.when(pl.program_id(0) == 0)
    def _():
        w = w_ref[...]
        w2 = jnp.sum(w * w, axis=1, keepdims=True)
        waug_ref[...] = jnp.concatenate(
            [w, w2, jnp.zeros((N_EMB, KAUG - DIM - 1), jnp.float32)], axis=1)

    x = x_ref[...]
    x2 = jnp.sum(x * x, axis=1, keepdims=True)
    x_aug = jnp.concatenate(
        [-2.0 * x, jnp.ones((BM, 1), jnp.float32),
         jnp.zeros((BM, KAUG - DIM - 1), jnp.float32)], axis=1)
    d = lax.dot_general(x_aug, waug_ref[...], (((1,), (1,)), ((), ())),
                        preferred_element_type=jnp.float32) + x2
    dist_ref[...] = d
    enc_ref[...] = jnp.argmin(d, axis=1).astype(jnp.int32)[:, None]
    part = jnp.sum(jnp.min(d, axis=1))

    @pl.when(pl.program_id(0) == 0)
    def _():
        loss_ref[...] = jnp.zeros_like(loss_ref)

    loss_ref[...] += jnp.full((1, 1), part, jnp.float32)


def _distances_enc_loss(flat_x, weight):
    return pl.pallas_call(
        _dist_kernel,
        grid=(N_BLOCKS,),
        in_specs=[
            pl.BlockSpec((BM, DIM), lambda i: (i, 0)),
            pl.BlockSpec((N_EMB, DIM), lambda i: (0, 0)),
        ],
        out_specs=[
            pl.BlockSpec((BM, N_EMB), lambda i: (i, 0)),
            pl.BlockSpec((BM, 1), lambda i: (i, 0)),
            pl.BlockSpec((1, 1), lambda i: (0, 0)),
        ],
        out_shape=[
            jax.ShapeDtypeStruct((ROWS, N_EMB), jnp.float32),
            jax.ShapeDtypeStruct((ROWS, 1), jnp.int32),
            jax.ShapeDtypeStruct((1, 1), jnp.float32),
        ],
        scratch_shapes=[pltpu.VMEM((N_EMB, KAUG), jnp.float32)],
    )(flat_x, weight)


def _sc_gather_body(w_hbm, enc_hbm, out_hbm, idx_v, rows_v, sem):
    wid = lax.axis_index("s") * SC_CORES + lax.axis_index("c")
    base = wid * B_PER_W
    pltpu.sync_copy(enc_hbm.at[pl.ds(wid * N_CHUNKS, N_CHUNKS)], idx_v)
    for j in range(N_CHUNKS):
        pltpu.async_copy(w_hbm.at[idx_v.at[j]],
                         rows_v.at[pl.ds(j * IDX_CHUNK, IDX_CHUNK)], sem).wait()
    pltpu.sync_copy(rows_v, out_hbm.at[pl.ds(base, B_PER_W)])


@functools.cache
def _sc_gather():
    return pl.kernel(
        _sc_gather_body,
        out_type=jax.ShapeDtypeStruct((ROWS, DIM_PAD), jnp.float32),
        scratch_types=[
            pltpu.VMEM((N_CHUNKS, IDX_CHUNK), jnp.int32),
            pltpu.VMEM((B_PER_W, DIM_PAD), jnp.float32),
            pltpu.SemaphoreType.DMA,
        ],
        mesh=plsc.VectorSubcoreMesh(core_axis_name="c", subcore_axis_name="s"),
    )


def kernel(input, weight):
    flat_x = jnp.transpose(input, (0, 2, 3, 1)).reshape(ROWS, DIM)
    distances, enc2d, loss_acc = _distances_enc_loss(flat_x, weight)
    encoding_flat = enc2d.reshape(ROWS)
    weight_pad = jnp.pad(weight, ((0, 0), (0, DIM_PAD - DIM)))
    quant_pad = _sc_gather()(weight_pad,
                             enc2d.reshape(ROWS // IDX_CHUNK, IDX_CHUNK))
    quant_flat = quant_pad[:, :DIM]
    quantized_st = jnp.transpose(
        quant_flat.reshape(8, 32, 32, DIM), (0, 3, 1, 2))
    encoding = encoding_flat.reshape(8, 32, 32)
    loss = loss_acc[0, 0] * (1.0 / input.size)
    return (quantized_st, encoding, distances, loss)
